# Initial kernel scaffold; baseline (speedup 1.0000x reference)
import jax
import jax.numpy as jnp
from jax import lax
from jax.experimental import pallas as pl
from jax.experimental.pallas import tpu as pltpu
from jax.experimental.pallas import tpu_sc as plsc

NFEATURES = 1_000_000
SIZE_HA = 32
BATCH = 16384
NFIELDS = 100

_INFO = plsc.get_sparse_core_info()
NC = _INFO.num_cores
NS = _INFO.num_subcores
NW = NC * NS
B_PER_W = BATCH // NW
CHUNK = 16
NCHUNKS = B_PER_W // CHUNK
HALF = 16


def _sc_body(phi_hbm, table_hbm, bias_hbm, out_hbm,
             idx_v, rows_v, out_v, bias_v, sem, osem):
    wid = lax.axis_index("s") * NC + lax.axis_index("c")
    base0 = wid * B_PER_W

    pltpu.sync_copy(bias_hbm, bias_v)
    b0 = bias_v[pl.ds(0, HALF)]
    b1 = bias_v[pl.ds(HALF, HALF)]

    def issue(g):
        pltpu.sync_copy(phi_hbm.at[pl.ds(base0 + g * CHUNK, CHUNK)], idx_v)
        for c in range(CHUNK):
            pltpu.async_copy(
                table_hbm.at[idx_v.at[c]],
                rows_v.at[pl.ds(c * NFIELDS, NFIELDS)],
                sem,
            )

    def drain():
        pltpu.make_async_copy(
            table_hbm.at[pl.ds(0, CHUNK * NFIELDS)], rows_v, sem
        ).wait()

    def reduce_chunk(g):
        def per_row(c, _):
            rbase = c * NFIELDS

            def per_field(f, accs):
                p0a, p0b, p1a, p1b, p2a, p2b, p3a, p3b = accs
                r = rbase + f * 4
                p0a = p0a + rows_v[r, pl.ds(0, HALF)]
                p0b = p0b + rows_v[r, pl.ds(HALF, HALF)]
                p1a = p1a + rows_v[r + 1, pl.ds(0, HALF)]
                p1b = p1b + rows_v[r + 1, pl.ds(HALF, HALF)]
                p2a = p2a + rows_v[r + 2, pl.ds(0, HALF)]
                p2b = p2b + rows_v[r + 2, pl.ds(HALF, HALF)]
                p3a = p3a + rows_v[r + 3, pl.ds(0, HALF)]
                p3b = p3b + rows_v[r + 3, pl.ds(HALF, HALF)]
                return p0a, p0b, p1a, p1b, p2a, p2b, p3a, p3b

            z = jnp.zeros((HALF,), jnp.float32)
            accs = lax.fori_loop(0, NFIELDS // 4, per_field, (z,) * 8)
            s0 = (accs[0] + accs[2]) + (accs[4] + accs[6]) + b0
            s1 = (accs[1] + accs[3]) + (accs[5] + accs[7]) + b1
            e0 = jnp.exp(s0 + s0)
            e1 = jnp.exp(s1 + s1)
            out_v[c, pl.ds(0, HALF)] = (e0 - 1.0) / (e0 + 1.0)
            out_v[c, pl.ds(HALF, HALF)] = (e1 - 1.0) / (e1 + 1.0)
            return 0

        lax.fori_loop(0, CHUNK, per_row, 0)
        pltpu.async_copy(out_v, out_hbm.at[pl.ds(base0 + g * CHUNK, CHUNK)],
                         osem)

    def round_body(g, _):
        issue(g)
        drain()
        reduce_chunk(g)
        pltpu.make_async_copy(
            out_v, out_hbm.at[pl.ds(0, CHUNK)], osem
        ).wait()
        return 0

    lax.fori_loop(0, NCHUNKS, round_body, 0)


@jax.jit
def _hmodel_sc(phi_a, table, bias):
    mesh = plsc.VectorSubcoreMesh(core_axis_name="c", subcore_axis_name="s")
    return pl.kernel(
        _sc_body,
        out_type=jax.ShapeDtypeStruct((BATCH, SIZE_HA), jnp.float32),
        mesh=mesh,
        scratch_types=[
            pltpu.VMEM((CHUNK, NFIELDS), jnp.int32),
            pltpu.VMEM((CHUNK * NFIELDS, SIZE_HA), jnp.float32),
            pltpu.VMEM((CHUNK, SIZE_HA), jnp.float32),
            pltpu.VMEM((SIZE_HA,), jnp.float32),
            pltpu.SemaphoreType.DMA,
            pltpu.SemaphoreType.DMA,
        ],
    )(phi_a, table, bias)


def kernel(phi_a, table, bias):
    return _hmodel_sc(phi_a.astype(jnp.int32), table, bias)


# speedup vs baseline: 9.2793x; 9.2793x over previous
import jax
import jax.numpy as jnp
from jax import lax
from jax.experimental import pallas as pl
from jax.experimental.pallas import tpu as pltpu
from jax.experimental.pallas import tpu_sc as plsc

NFEATURES = 1_000_000
SIZE_HA = 32
BATCH = 16384
NFIELDS = 100

_INFO = plsc.get_sparse_core_info()
NC = _INFO.num_cores
NS = _INFO.num_subcores
NW = NC * NS
B_PER_W = BATCH // NW
CHUNK = 16
NCHUNKS = B_PER_W // CHUNK
HALF = 16


def _sc_body(phi_hbm, table_hbm, bias_hbm, out_hbm,
             idx_v, rows_v, out_v, bias_v, sem, osem):
    wid = lax.axis_index("s") * NC + lax.axis_index("c")
    base0 = wid * B_PER_W

    pltpu.sync_copy(bias_hbm, bias_v)
    b0 = bias_v[pl.ds(0, HALF)]
    b1 = bias_v[pl.ds(HALF, HALF)]

    def issue(g):
        pltpu.sync_copy(phi_hbm.at[pl.ds(base0 + g * CHUNK, CHUNK)], idx_v)
        for c in range(CHUNK):
            pltpu.async_copy(
                table_hbm.at[idx_v.at[c]],
                rows_v.at[pl.ds(c * NFIELDS, NFIELDS)],
                sem,
            )

    def drain():
        pltpu.make_async_copy(
            table_hbm.at[pl.ds(0, CHUNK * NFIELDS)], rows_v, sem
        ).wait()

    def reduce_chunk(g):
        def per_row(c, _):
            rbase = c * NFIELDS

            def per_field(f, accs):
                p0a, p0b, p1a, p1b, p2a, p2b, p3a, p3b = accs
                r = rbase + f * 4
                p0a = p0a + rows_v[r, pl.ds(0, HALF)]
                p0b = p0b + rows_v[r, pl.ds(HALF, HALF)]
                p1a = p1a + rows_v[r + 1, pl.ds(0, HALF)]
                p1b = p1b + rows_v[r + 1, pl.ds(HALF, HALF)]
                p2a = p2a + rows_v[r + 2, pl.ds(0, HALF)]
                p2b = p2b + rows_v[r + 2, pl.ds(HALF, HALF)]
                p3a = p3a + rows_v[r + 3, pl.ds(0, HALF)]
                p3b = p3b + rows_v[r + 3, pl.ds(HALF, HALF)]
                return p0a, p0b, p1a, p1b, p2a, p2b, p3a, p3b

            z = jnp.zeros((HALF,), jnp.float32)
            accs = lax.fori_loop(0, NFIELDS // 4, per_field, (z,) * 8)
            s0 = (accs[0] + accs[2]) + (accs[4] + accs[6]) + b0
            s1 = (accs[1] + accs[3]) + (accs[5] + accs[7]) + b1
            e0 = jnp.exp(s0 + s0)
            e1 = jnp.exp(s1 + s1)
            out_v[c, pl.ds(0, HALF)] = (e0 - 1.0) / (e0 + 1.0)
            out_v[c, pl.ds(HALF, HALF)] = (e1 - 1.0) / (e1 + 1.0)
            return 0

        lax.fori_loop(0, CHUNK, per_row, 0)
        pltpu.async_copy(out_v, out_hbm.at[pl.ds(base0 + g * CHUNK, CHUNK)],
                         osem)

    def round_body(g, _):
        issue(g)
        drain()
        reduce_chunk(g)
        pltpu.make_async_copy(
            out_v, out_hbm.at[pl.ds(0, CHUNK)], osem
        ).wait()
        return 0

    lax.fori_loop(0, NCHUNKS, round_body, 0)


@jax.jit
def _hmodel_sc(phi_a, table, bias):
    mesh = plsc.VectorSubcoreMesh(core_axis_name="c", subcore_axis_name="s")
    return pl.kernel(
        _sc_body,
        out_type=jax.ShapeDtypeStruct((BATCH, SIZE_HA), jnp.float32),
        mesh=mesh,
        compiler_params=pltpu.CompilerParams(use_tc_tiling_on_sc=False),
        scratch_types=[
            pltpu.VMEM((CHUNK, NFIELDS), jnp.int32),
            pltpu.VMEM((CHUNK * NFIELDS, SIZE_HA), jnp.float32),
            pltpu.VMEM((CHUNK, SIZE_HA), jnp.float32),
            pltpu.VMEM((SIZE_HA,), jnp.float32),
            pltpu.SemaphoreType.DMA,
            pltpu.SemaphoreType.DMA,
        ],
    )(phi_a, table, bias)


def kernel(phi_a, table, bias):
    return _hmodel_sc(phi_a.astype(jnp.int32), table, bias)


# speedup vs baseline: 10.4211x; 1.1230x over previous
import jax
import jax.numpy as jnp
from jax import lax
from jax.experimental import pallas as pl
from jax.experimental.pallas import tpu as pltpu
from jax.experimental.pallas import tpu_sc as plsc

NFEATURES = 1_000_000
SIZE_HA = 32
BATCH = 16384
NFIELDS = 100

_INFO = plsc.get_sparse_core_info()
NC = _INFO.num_cores
NS = _INFO.num_subcores
NW = NC * NS
B_PER_W = BATCH // NW
CHUNK = 16
NCHUNKS = B_PER_W // CHUNK
HALF = 16


def _sc_body(phi_hbm, table_hbm, bias_hbm, out_hbm,
             idx0, idx1, rows0, rows1, out0, out1, bias_v,
             sem0, sem1, osem0, osem1):
    wid = lax.axis_index("s") * NC + lax.axis_index("c")
    base0 = wid * B_PER_W

    pltpu.sync_copy(bias_hbm, bias_v)
    b0 = bias_v[pl.ds(0, HALF)]
    b1 = bias_v[pl.ds(HALF, HALF)]

    def issue(g, idx_v, rows_v, sem):
        pltpu.sync_copy(phi_hbm.at[pl.ds(base0 + g * CHUNK, CHUNK)], idx_v)
        for c in range(CHUNK):
            pltpu.async_copy(
                table_hbm.at[idx_v.at[c]],
                rows_v.at[pl.ds(c * NFIELDS, NFIELDS)],
                sem,
            )

    def drain(rows_v, sem):
        pltpu.make_async_copy(
            table_hbm.at[pl.ds(0, CHUNK * NFIELDS)], rows_v, sem
        ).wait()

    def reduce_chunk(g, rows_v, out_v, osem):
        def per_row(c, _):
            rbase = c * NFIELDS

            def per_field(f, accs):
                p0a, p0b, p1a, p1b, p2a, p2b, p3a, p3b = accs
                r = rbase + f * 4
                p0a = p0a + rows_v[r, pl.ds(0, HALF)]
                p0b = p0b + rows_v[r, pl.ds(HALF, HALF)]
                p1a = p1a + rows_v[r + 1, pl.ds(0, HALF)]
                p1b = p1b + rows_v[r + 1, pl.ds(HALF, HALF)]
                p2a = p2a + rows_v[r + 2, pl.ds(0, HALF)]
                p2b = p2b + rows_v[r + 2, pl.ds(HALF, HALF)]
                p3a = p3a + rows_v[r + 3, pl.ds(0, HALF)]
                p3b = p3b + rows_v[r + 3, pl.ds(HALF, HALF)]
                return p0a, p0b, p1a, p1b, p2a, p2b, p3a, p3b

            z = jnp.zeros((HALF,), jnp.float32)
            accs = lax.fori_loop(0, NFIELDS // 4, per_field, (z,) * 8)
            s0 = (accs[0] + accs[2]) + (accs[4] + accs[6]) + b0
            s1 = (accs[1] + accs[3]) + (accs[5] + accs[7]) + b1
            e0 = jnp.exp(s0 + s0)
            e1 = jnp.exp(s1 + s1)
            out_v[c, pl.ds(0, HALF)] = (e0 - 1.0) / (e0 + 1.0)
            out_v[c, pl.ds(HALF, HALF)] = (e1 - 1.0) / (e1 + 1.0)
            return 0

        lax.fori_loop(0, CHUNK, per_row, 0)
        pltpu.async_copy(out_v, out_hbm.at[pl.ds(base0 + g * CHUNK, CHUNK)],
                         osem)

    def drain_out(out_v, osem):
        pltpu.make_async_copy(out_v, out_hbm.at[pl.ds(0, CHUNK)], osem).wait()

    issue(0, idx0, rows0, sem0)

    def pair_body(t, _):
        g0 = 2 * t
        issue(g0 + 1, idx1, rows1, sem1)
        drain(rows0, sem0)
        reduce_chunk(g0, rows0, out0, osem0)

        @pl.when(t + 1 < NCHUNKS // 2)
        def _():
            issue(g0 + 2, idx0, rows0, sem0)

        drain(rows1, sem1)
        reduce_chunk(g0 + 1, rows1, out1, osem1)
        drain_out(out0, osem0)
        drain_out(out1, osem1)
        return 0

    lax.fori_loop(0, NCHUNKS // 2, pair_body, 0)


@jax.jit
def _hmodel_sc(phi_a, table, bias):
    mesh = plsc.VectorSubcoreMesh(core_axis_name="c", subcore_axis_name="s")
    return pl.kernel(
        _sc_body,
        out_type=jax.ShapeDtypeStruct((BATCH, SIZE_HA), jnp.float32),
        mesh=mesh,
        compiler_params=pltpu.CompilerParams(use_tc_tiling_on_sc=False),
        scratch_types=[
            pltpu.VMEM((CHUNK, NFIELDS), jnp.int32),
            pltpu.VMEM((CHUNK, NFIELDS), jnp.int32),
            pltpu.VMEM((CHUNK * NFIELDS, SIZE_HA), jnp.float32),
            pltpu.VMEM((CHUNK * NFIELDS, SIZE_HA), jnp.float32),
            pltpu.VMEM((CHUNK, SIZE_HA), jnp.float32),
            pltpu.VMEM((CHUNK, SIZE_HA), jnp.float32),
            pltpu.VMEM((SIZE_HA,), jnp.float32),
            pltpu.SemaphoreType.DMA,
            pltpu.SemaphoreType.DMA,
            pltpu.SemaphoreType.DMA,
            pltpu.SemaphoreType.DMA,
        ],
    )(phi_a, table, bias)


def kernel(phi_a, table, bias):
    return _hmodel_sc(phi_a.astype(jnp.int32), table, bias)


# speedup vs baseline: 11.7961x; 1.1319x over previous
import jax
import jax.numpy as jnp
from jax import lax
from jax.experimental import pallas as pl
from jax.experimental.pallas import tpu as pltpu
from jax.experimental.pallas import tpu_sc as plsc

NFEATURES = 1_000_000
SIZE_HA = 32
BATCH = 16384
NFIELDS = 100

_INFO = plsc.get_sparse_core_info()
NC = _INFO.num_cores
NS = _INFO.num_subcores
NW = NC * NS
B_PER_W = BATCH // NW
CHUNK = 16
NCHUNKS = B_PER_W // CHUNK
HALF = 16


def _sc_body(phi_hbm, table_hbm, bias_hbm, out_hbm,
             idx0, idx1, rows0, rows1, out0, out1, bias_v,
             sem0, sem1, osem0, osem1):
    wid = lax.axis_index("s") * NC + lax.axis_index("c")
    base0 = wid * B_PER_W

    pltpu.sync_copy(bias_hbm, bias_v)
    b0 = bias_v[pl.ds(0, HALF)]
    b1 = bias_v[pl.ds(HALF, HALF)]

    def issue(g, idx_v, rows_v, sem):
        pltpu.sync_copy(phi_hbm.at[pl.ds(base0 + g * CHUNK, CHUNK)], idx_v)
        for c in range(CHUNK):
            pltpu.async_copy(
                table_hbm.at[idx_v.at[c]],
                rows_v.at[pl.ds(c * NFIELDS, NFIELDS)],
                sem,
            )

    def drain(rows_v, sem):
        pltpu.make_async_copy(
            table_hbm.at[pl.ds(0, CHUNK * NFIELDS)], rows_v, sem
        ).wait()

    def reduce_chunk(g, rows_v, out_v, osem):
        def per_row(c, _):
            rbase = c * NFIELDS

            def per_field(f, accs):
                p0a, p0b, p1a, p1b, p2a, p2b, p3a, p3b = accs
                r = rbase + f * 4
                p0a = p0a + rows_v[r, pl.ds(0, HALF)]
                p0b = p0b + rows_v[r, pl.ds(HALF, HALF)]
                p1a = p1a + rows_v[r + 1, pl.ds(0, HALF)]
                p1b = p1b + rows_v[r + 1, pl.ds(HALF, HALF)]
                p2a = p2a + rows_v[r + 2, pl.ds(0, HALF)]
                p2b = p2b + rows_v[r + 2, pl.ds(HALF, HALF)]
                p3a = p3a + rows_v[r + 3, pl.ds(0, HALF)]
                p3b = p3b + rows_v[r + 3, pl.ds(HALF, HALF)]
                return p0a, p0b, p1a, p1b, p2a, p2b, p3a, p3b

            z = jnp.zeros((HALF,), jnp.float32)
            accs = lax.fori_loop(0, NFIELDS // 4, per_field, (z,) * 8)
            s0 = (accs[0] + accs[2]) + (accs[4] + accs[6]) + b0
            s1 = (accs[1] + accs[3]) + (accs[5] + accs[7]) + b1
            e0 = jnp.exp(s0 + s0)
            e1 = jnp.exp(s1 + s1)
            out_v[c, pl.ds(0, HALF)] = (e0 - 1.0) / (e0 + 1.0)
            out_v[c, pl.ds(HALF, HALF)] = (e1 - 1.0) / (e1 + 1.0)
            return 0

        lax.fori_loop(0, CHUNK, per_row, 0)
        pltpu.async_copy(out_v, out_hbm.at[pl.ds(base0 + g * CHUNK, CHUNK)],
                         osem)

    def drain_out(out_v, osem):
        pltpu.make_async_copy(out_v, out_hbm.at[pl.ds(0, CHUNK)], osem).wait()

    issue(0, idx0, rows0, sem0)

    def pair_body(t, _):
        g0 = 2 * t
        issue(g0 + 1, idx1, rows1, sem1)
        drain(rows0, sem0)
        reduce_chunk(g0, rows0, out0, osem0)

        @pl.when(t + 1 < NCHUNKS // 2)
        def _():
            issue(g0 + 2, idx0, rows0, sem0)

        drain(rows1, sem1)
        reduce_chunk(g0 + 1, rows1, out1, osem1)
        drain_out(out0, osem0)
        drain_out(out1, osem1)
        return 0

    lax.fori_loop(0, NCHUNKS // 2, pair_body, 0)


NROWS = NFEATURES + 1
TPAD_ROWS = 1_000_008
T1D_LEN = TPAD_ROWS * SIZE_HA
TCOLS = 512
NFULL = NROWS // TCOLS
TAIL = NROWS - NFULL * TCOLS
NSTEPS = NFULL


SWROW = TCOLS + HALF
SWOFF = HALF
SWLEN = HALF * SWROW * 2 + HALF
NDIAG = TCOLS + HALF
OUTOFF = 512
OUTLEN = OUTOFF + TCOLS * SIZE_HA + OUTOFF - 32


def _transpose_body(tt_hbm, tail_hbm, t1d_hbm, in0, in1, out0, out1, sw,
                    tail_v, isem0, isem1, osem0, osem1):
    wid = lax.axis_index("s") * NC + lax.axis_index("c")
    gdiag = lax.iota(jnp.int32, HALF) * (SWROW + 1)
    sdiag = lax.iota(jnp.int32, HALF) * (SIZE_HA + 1)

    def issue_in(k, in_v, sem):
        pltpu.async_copy(tt_hbm.at[:, pl.ds(k * TCOLS, TCOLS)], in_v, sem)

    def transpose(in_v, out_v):
        def per_c(c, _):
            def per_j(j, _):
                sw[pl.ds(c * SWROW + SWOFF + j * HALF, HALF)] = (
                    in_v[c, pl.ds(j * HALF, HALF)])
                return 0

            lax.fori_loop(0, TCOLS // HALF, per_j, 0, unroll=4)
            return 0

        lax.fori_loop(0, SIZE_HA, per_c, 0)

        def per_q(q, carry):
            g0, g1, g2, g3, s0, s1, s2, s3 = carry
            for gi, si in ((g0, s0), (g1, s1), (g2, s2), (g3, s3)):
                v_lo = plsc.load_gather(sw, [gi])
                v_hi = plsc.load_gather(sw, [gi + HALF * SWROW])
                plsc.store_scatter(out_v, [si], v_lo)
                plsc.store_scatter(out_v, [si + HALF], v_hi)
            return (g0 + 4, g1 + 4, g2 + 4, g3 + 4,
                    s0 + 4 * SIZE_HA, s1 + 4 * SIZE_HA, s2 + 4 * SIZE_HA,
                    s3 + 4 * SIZE_HA)

        lax.fori_loop(
            0, NDIAG // 4, per_q,
            (gdiag, gdiag + 1, gdiag + 2, gdiag + 3,
             sdiag, sdiag + SIZE_HA, sdiag + 2 * SIZE_HA,
             sdiag + 3 * SIZE_HA))

    def handle(i, in_v, out_v, isem, osem):
        k = wid + i * NW
        pltpu.make_async_copy(tt_hbm.at[:, pl.ds(0, TCOLS)], in_v,
                              isem).wait()
        transpose(in_v, out_v)
        pltpu.async_copy(out_v.at[pl.ds(OUTOFF, TCOLS * SIZE_HA)],
                         t1d_hbm.at[pl.ds(k * TCOLS * SIZE_HA,
                                          TCOLS * SIZE_HA)],
                         osem)
        pltpu.make_async_copy(out_v.at[pl.ds(OUTOFF, TCOLS * SIZE_HA)],
                              t1d_hbm.at[pl.ds(0, TCOLS * SIZE_HA)],
                              osem).wait()

    @pl.when(wid == 0)
    def _():
        pltpu.sync_copy(tail_hbm, tail_v)
        pltpu.sync_copy(tail_v, t1d_hbm.at[pl.ds(NFULL * TCOLS * SIZE_HA,
                                                 TAIL * SIZE_HA)])

    my_steps = (NSTEPS - 1 - wid) // NW + 1
    issue_in(wid, in0, isem0)
    MAXSTEPS = -(-NSTEPS // NW)

    def pair_body(t, _):
        i0 = 2 * t
        i1 = i0 + 1

        @pl.when(i1 < my_steps)
        def _():
            issue_in(wid + i1 * NW, in1, isem1)

        @pl.when(i0 < my_steps)
        def _():
            handle(i0, in0, out0, isem0, osem0)

        @pl.when(i0 + 2 < my_steps)
        def _():
            issue_in(wid + (i0 + 2) * NW, in0, isem0)

        @pl.when(i1 < my_steps)
        def _():
            handle(i1, in1, out1, isem1, osem1)

        return 0

    lax.fori_loop(0, (MAXSTEPS + 1) // 2, pair_body, 0)


@jax.jit
def _hmodel_sc(phi_a, table, bias):
    mesh = plsc.VectorSubcoreMesh(core_axis_name="c", subcore_axis_name="s")
    tail = jnp.ravel(lax.slice(table, (NFULL * TCOLS, 0), (NROWS, SIZE_HA)))
    t1d = pl.kernel(
        _transpose_body,
        out_type=jax.ShapeDtypeStruct((T1D_LEN,), jnp.float32),
        mesh=mesh,
        compiler_params=pltpu.CompilerParams(use_tc_tiling_on_sc=True,
                                             needs_layout_passes=False),
        scratch_types=[
            pltpu.VMEM((SIZE_HA, TCOLS), jnp.float32),
            pltpu.VMEM((SIZE_HA, TCOLS), jnp.float32),
            pltpu.VMEM((OUTLEN,), jnp.float32),
            pltpu.VMEM((OUTLEN,), jnp.float32),
            pltpu.VMEM((SWLEN,), jnp.float32),
            pltpu.VMEM((TAIL * SIZE_HA,), jnp.float32),
            pltpu.SemaphoreType.DMA,
            pltpu.SemaphoreType.DMA,
            pltpu.SemaphoreType.DMA,
            pltpu.SemaphoreType.DMA,
        ],
    )(table.T, tail)
    t2 = t1d.reshape(TPAD_ROWS, SIZE_HA)
    return pl.kernel(
        _sc_body,
        out_type=jax.ShapeDtypeStruct((BATCH, SIZE_HA), jnp.float32),
        mesh=mesh,
        compiler_params=pltpu.CompilerParams(use_tc_tiling_on_sc=False),
        scratch_types=[
            pltpu.VMEM((CHUNK, NFIELDS), jnp.int32),
            pltpu.VMEM((CHUNK, NFIELDS), jnp.int32),
            pltpu.VMEM((CHUNK * NFIELDS, SIZE_HA), jnp.float32),
            pltpu.VMEM((CHUNK * NFIELDS, SIZE_HA), jnp.float32),
            pltpu.VMEM((CHUNK, SIZE_HA), jnp.float32),
            pltpu.VMEM((CHUNK, SIZE_HA), jnp.float32),
            pltpu.VMEM((SIZE_HA,), jnp.float32),
            pltpu.SemaphoreType.DMA,
            pltpu.SemaphoreType.DMA,
            pltpu.SemaphoreType.DMA,
            pltpu.SemaphoreType.DMA,
        ],
    )(phi_a, t2, bias)


def kernel(phi_a, table, bias):
    return _hmodel_sc(phi_a.astype(jnp.int32), table, bias)


# speedup vs baseline: 16.8559x; 1.4289x over previous
import jax
import jax.numpy as jnp
from jax import lax
from jax.experimental import pallas as pl
from jax.experimental.pallas import tpu as pltpu
from jax.experimental.pallas import tpu_sc as plsc

NFEATURES = 1_000_000
SIZE_HA = 32
BATCH = 16384
NFIELDS = 100

_INFO = plsc.get_sparse_core_info()
NC = _INFO.num_cores
NS = _INFO.num_subcores
NW = NC * NS
B_PER_W = BATCH // NW
CHUNK = 16
NCHUNKS = B_PER_W // CHUNK
HALF = 16


def _sc_body(phi_hbm, table_hbm, bias_hbm, out_hbm,
             idx0, idx1, rows0, rows1, out0, out1, bias_v,
             sem0, sem1, osem0, osem1):
    wid = lax.axis_index("s") * NC + lax.axis_index("c")
    base0 = wid * B_PER_W

    pltpu.sync_copy(bias_hbm, bias_v)
    b0 = bias_v[pl.ds(0, HALF)]
    b1 = bias_v[pl.ds(HALF, HALF)]

    def issue(g, idx_v, rows_v, sem):
        pltpu.sync_copy(phi_hbm.at[pl.ds(base0 + g * CHUNK, CHUNK)], idx_v)
        for c in range(CHUNK):
            pltpu.async_copy(
                table_hbm.at[idx_v.at[c]],
                rows_v.at[pl.ds(c * NFIELDS, NFIELDS)],
                sem,
            )

    def drain(rows_v, sem):
        pltpu.make_async_copy(
            table_hbm.at[pl.ds(0, CHUNK * NFIELDS)], rows_v, sem
        ).wait()

    def reduce_chunk(g, rows_v, out_v, osem):
        def per_row(c, _):
            rbase = c * NFIELDS

            def per_field(f, accs):
                p0a, p0b, p1a, p1b, p2a, p2b, p3a, p3b = accs
                r = rbase + f * 4
                p0a = p0a + rows_v[r, pl.ds(0, HALF)]
                p0b = p0b + rows_v[r, pl.ds(HALF, HALF)]
                p1a = p1a + rows_v[r + 1, pl.ds(0, HALF)]
                p1b = p1b + rows_v[r + 1, pl.ds(HALF, HALF)]
                p2a = p2a + rows_v[r + 2, pl.ds(0, HALF)]
                p2b = p2b + rows_v[r + 2, pl.ds(HALF, HALF)]
                p3a = p3a + rows_v[r + 3, pl.ds(0, HALF)]
                p3b = p3b + rows_v[r + 3, pl.ds(HALF, HALF)]
                return p0a, p0b, p1a, p1b, p2a, p2b, p3a, p3b

            z = jnp.zeros((HALF,), jnp.float32)
            accs = lax.fori_loop(0, NFIELDS // 4, per_field, (z,) * 8)
            s0 = (accs[0] + accs[2]) + (accs[4] + accs[6]) + b0
            s1 = (accs[1] + accs[3]) + (accs[5] + accs[7]) + b1
            e0 = jnp.exp(s0 + s0)
            e1 = jnp.exp(s1 + s1)
            out_v[c, pl.ds(0, HALF)] = (e0 - 1.0) / (e0 + 1.0)
            out_v[c, pl.ds(HALF, HALF)] = (e1 - 1.0) / (e1 + 1.0)
            return 0

        lax.fori_loop(0, CHUNK, per_row, 0)
        pltpu.async_copy(out_v, out_hbm.at[pl.ds(base0 + g * CHUNK, CHUNK)],
                         osem)

    def drain_out(out_v, osem):
        pltpu.make_async_copy(out_v, out_hbm.at[pl.ds(0, CHUNK)], osem).wait()

    issue(0, idx0, rows0, sem0)

    def pair_body(t, _):
        g0 = 2 * t
        issue(g0 + 1, idx1, rows1, sem1)
        drain(rows0, sem0)
        reduce_chunk(g0, rows0, out0, osem0)

        @pl.when(t + 1 < NCHUNKS // 2)
        def _():
            issue(g0 + 2, idx0, rows0, sem0)

        drain(rows1, sem1)
        reduce_chunk(g0 + 1, rows1, out1, osem1)
        drain_out(out0, osem0)
        drain_out(out1, osem1)
        return 0

    lax.fori_loop(0, NCHUNKS // 2, pair_body, 0)


NROWS = NFEATURES + 1
TPAD_ROWS = 1_000_008
T1D_LEN = TPAD_ROWS * SIZE_HA
TCOLS = 512
NFULL = NROWS // TCOLS
TAIL = NROWS - NFULL * TCOLS
NSTEPS = NFULL


SWROW = TCOLS + HALF
SWOFF = HALF
SWLEN = HALF * SWROW * 2 + HALF
NDIAG = TCOLS + HALF
OUTOFF = 512
OUTLEN = OUTOFF + TCOLS * SIZE_HA + OUTOFF - 32


def _transpose_body(tt_hbm, tail_hbm, t1d_hbm, in0, in1, out0, out1, sw,
                    tail_v, isem0, isem1, osem0, osem1):
    wid = lax.axis_index("s") * NC + lax.axis_index("c")
    gdiag = lax.iota(jnp.int32, HALF) * (SWROW + 1)
    sdiag = lax.iota(jnp.int32, HALF) * (SIZE_HA + 1)

    def issue_in(k, in_v, sem):
        pltpu.async_copy(tt_hbm.at[:, pl.ds(k * TCOLS, TCOLS)], in_v, sem)

    def transpose(in_v, out_v):
        def per_c(c, _):
            @plsc.parallel_loop(0, TCOLS // HALF, unroll=8)
            def _copy(j):
                sw[pl.ds(c * SWROW + SWOFF + j * HALF, HALF)] = (
                    in_v[c, pl.ds(j * HALF, HALF)])

            return 0

        lax.fori_loop(0, SIZE_HA, per_c, 0)

        def per_q(q, carry):
            g0, g1, g2, g3, s0, s1, s2, s3 = carry
            for gi, si in ((g0, s0), (g1, s1), (g2, s2), (g3, s3)):
                v_lo = plsc.load_gather(sw, [gi])
                v_hi = plsc.load_gather(sw, [gi + HALF * SWROW])
                plsc.store_scatter(out_v, [si], v_lo)
                plsc.store_scatter(out_v, [si + HALF], v_hi)
            return (g0 + 4, g1 + 4, g2 + 4, g3 + 4,
                    s0 + 4 * SIZE_HA, s1 + 4 * SIZE_HA, s2 + 4 * SIZE_HA,
                    s3 + 4 * SIZE_HA)

        lax.fori_loop(
            0, NDIAG // 4, per_q,
            (gdiag, gdiag + 1, gdiag + 2, gdiag + 3,
             sdiag, sdiag + SIZE_HA, sdiag + 2 * SIZE_HA,
             sdiag + 3 * SIZE_HA),
            unroll=2)

    def handle(i, in_v, out_v, isem, osem):
        k = wid + i * NW
        pltpu.make_async_copy(tt_hbm.at[:, pl.ds(0, TCOLS)], in_v,
                              isem).wait()
        transpose(in_v, out_v)
        pltpu.async_copy(out_v.at[pl.ds(OUTOFF, TCOLS * SIZE_HA)],
                         t1d_hbm.at[pl.ds(k * TCOLS * SIZE_HA,
                                          TCOLS * SIZE_HA)],
                         osem)
        pltpu.make_async_copy(out_v.at[pl.ds(OUTOFF, TCOLS * SIZE_HA)],
                              t1d_hbm.at[pl.ds(0, TCOLS * SIZE_HA)],
                              osem).wait()

    @pl.when(wid == 0)
    def _():
        pltpu.sync_copy(tail_hbm, tail_v)
        pltpu.sync_copy(tail_v, t1d_hbm.at[pl.ds(NFULL * TCOLS * SIZE_HA,
                                                 TAIL * SIZE_HA)])

    my_steps = (NSTEPS - 1 - wid) // NW + 1
    issue_in(wid, in0, isem0)
    MAXSTEPS = -(-NSTEPS // NW)

    def pair_body(t, _):
        i0 = 2 * t
        i1 = i0 + 1

        @pl.when(i1 < my_steps)
        def _():
            issue_in(wid + i1 * NW, in1, isem1)

        @pl.when(i0 < my_steps)
        def _():
            handle(i0, in0, out0, isem0, osem0)

        @pl.when(i0 + 2 < my_steps)
        def _():
            issue_in(wid + (i0 + 2) * NW, in0, isem0)

        @pl.when(i1 < my_steps)
        def _():
            handle(i1, in1, out1, isem1, osem1)

        return 0

    lax.fori_loop(0, (MAXSTEPS + 1) // 2, pair_body, 0)


@jax.jit
def _hmodel_sc(phi_a, table, bias):
    mesh = plsc.VectorSubcoreMesh(core_axis_name="c", subcore_axis_name="s")
    tail = jnp.ravel(lax.slice(table, (NFULL * TCOLS, 0), (NROWS, SIZE_HA)))
    t1d = pl.kernel(
        _transpose_body,
        out_type=jax.ShapeDtypeStruct((T1D_LEN,), jnp.float32),
        mesh=mesh,
        compiler_params=pltpu.CompilerParams(use_tc_tiling_on_sc=True,
                                             needs_layout_passes=False),
        scratch_types=[
            pltpu.VMEM((SIZE_HA, TCOLS), jnp.float32),
            pltpu.VMEM((SIZE_HA, TCOLS), jnp.float32),
            pltpu.VMEM((OUTLEN,), jnp.float32),
            pltpu.VMEM((OUTLEN,), jnp.float32),
            pltpu.VMEM((SWLEN,), jnp.float32),
            pltpu.VMEM((TAIL * SIZE_HA,), jnp.float32),
            pltpu.SemaphoreType.DMA,
            pltpu.SemaphoreType.DMA,
            pltpu.SemaphoreType.DMA,
            pltpu.SemaphoreType.DMA,
        ],
    )(table.T, tail)
    t2 = t1d.reshape(TPAD_ROWS, SIZE_HA)
    return pl.kernel(
        _sc_body,
        out_type=jax.ShapeDtypeStruct((BATCH, SIZE_HA), jnp.float32),
        mesh=mesh,
        compiler_params=pltpu.CompilerParams(use_tc_tiling_on_sc=False),
        scratch_types=[
            pltpu.VMEM((CHUNK, NFIELDS), jnp.int32),
            pltpu.VMEM((CHUNK, NFIELDS), jnp.int32),
            pltpu.VMEM((CHUNK * NFIELDS, SIZE_HA), jnp.float32),
            pltpu.VMEM((CHUNK * NFIELDS, SIZE_HA), jnp.float32),
            pltpu.VMEM((CHUNK, SIZE_HA), jnp.float32),
            pltpu.VMEM((CHUNK, SIZE_HA), jnp.float32),
            pltpu.VMEM((SIZE_HA,), jnp.float32),
            pltpu.SemaphoreType.DMA,
            pltpu.SemaphoreType.DMA,
            pltpu.SemaphoreType.DMA,
            pltpu.SemaphoreType.DMA,
        ],
    )(phi_a, t2, bias)


def kernel(phi_a, table, bias):
    return _hmodel_sc(phi_a.astype(jnp.int32), table, bias)
